# baseline (speedup 1.0000x reference)
import jax
import jax.numpy as jnp
from jax.experimental import pallas as pl
from jax.experimental.pallas import tpu as pltpu
from jax.sharding import PartitionSpec as P

B, C, L = 64, 1, 1632
H, D, K = 64, 512, 1024
T = L // 4
N = B * T
BLK = 512


def _vq_kernel(img4_ref, W1e_ref, b1e_ref, W2e_ref, b2e_ref, w3e_ref, b3e_ref,
               cbT_ref, cbh_ref, w1d_ref, b1d_ref, W2d_ref, b2d_ref,
               W3d_ref, b3d_ref, out4_ref, idx_ref, q_ref, commit_ref,
               recon_ref, c2_ref):
    step = pl.program_id(0)

    @pl.when(step == 0)
    def _init():
        cb = cbh_ref[...]
        c2col = jnp.sum(cb * cb, axis=1, keepdims=True)
        c2_ref[...] = c2col.T
        commit_ref[...] = jnp.zeros_like(commit_ref)
        recon_ref[...] = jnp.zeros_like(recon_ref)

    img4 = img4_ref[...]
    h12 = jax.nn.relu(jnp.dot(img4, W1e_ref[...], preferred_element_type=jnp.float32)
                      + b1e_ref[...])
    h2 = jax.nn.relu(jnp.dot(h12, W2e_ref[...], preferred_element_type=jnp.float32)
                     + b2e_ref[...])
    x = jnp.dot(h2, w3e_ref[...], preferred_element_type=jnp.float32) + b3e_ref[...]

    x2 = jnp.sum(x * x, axis=1, keepdims=True)
    scores = jnp.dot(x, cbT_ref[...], preferred_element_type=jnp.float32)
    dist = x2 - 2.0 * scores + c2_ref[...]
    m = jnp.min(dist, axis=1, keepdims=True)
    iota = jax.lax.broadcasted_iota(jnp.int32, dist.shape, 1)
    idx = jnp.min(jnp.where(dist == m, iota, K), axis=1, keepdims=True)
    idx_ref[...] = idx

    onehot = (iota == idx).astype(jnp.float32)
    q = jnp.dot(onehot, cbh_ref[...], preferred_element_type=jnp.float32)
    q_ref[...] = q

    h1d = jax.nn.relu(jnp.dot(q, w1d_ref[...], preferred_element_type=jnp.float32)
                      + b1d_ref[...])
    h2d = jax.nn.relu(jnp.dot(h1d, W2d_ref[...], preferred_element_type=jnp.float32)
                      + b2d_ref[...])
    out4 = jnp.dot(h2d, W3d_ref[...], preferred_element_type=jnp.float32) + b3d_ref[...]
    out4_ref[...] = out4

    commit_ref[...] += jnp.sum(m).reshape(1, 1)
    diff = img4 - out4
    recon_ref[...] += jnp.sum(diff * diff).reshape(1, 1)


def _run_shard(img4, W1e, b1e, W2e, b2e, w3e, b3e, cbT, cbh, w1d, b1d,
               W2d, b2d, W3d, b3d):
    n = img4.shape[0]
    f32 = jnp.float32
    full = lambda shape: pl.BlockSpec(shape, lambda i: tuple(0 for _ in shape))
    return pl.pallas_call(
        _vq_kernel,
        grid=(n // BLK,),
        in_specs=[
            pl.BlockSpec((BLK, 4), lambda i: (i, 0)),
            full((4, 2 * H)), full((1, 2 * H)), full((2 * H, H)), full((1, H)),
            full((H, D)), full((1, D)), full((D, K)), full((K, D)),
            full((D, 2 * H)), full((1, 2 * H)), full((2 * H, 4 * H)),
            full((1, 4 * H)), full((4 * H, 4)), full((1, 1)),
        ],
        out_specs=[
            pl.BlockSpec((BLK, 4), lambda i: (i, 0)),
            pl.BlockSpec((BLK, 1), lambda i: (i, 0)),
            pl.BlockSpec((BLK, D), lambda i: (i, 0)),
            pl.BlockSpec((1, 1), lambda i: (0, 0)),
            pl.BlockSpec((1, 1), lambda i: (0, 0)),
        ],
        out_shape=[
            jax.ShapeDtypeStruct((n, 4), f32),
            jax.ShapeDtypeStruct((n, 1), jnp.int32),
            jax.ShapeDtypeStruct((n, D), f32),
            jax.ShapeDtypeStruct((1, 1), f32),
            jax.ShapeDtypeStruct((1, 1), f32),
        ],
        scratch_shapes=[pltpu.VMEM((1, K), f32)],
    )(img4, W1e, b1e, W2e, b2e, w3e, b3e, cbT, cbh, w1d, b1d, W2d, b2d,
      W3d, b3d)


@jax.jit
def kernel(img, enc_w1, enc_b1, enc_w2, enc_b2, enc_w3, enc_b3, codebook,
           dec_w1, dec_b1, dec_w2, dec_b2, dec_w3, dec_b3):
    f32 = jnp.float32
    img4 = img.reshape(B, T, 4).reshape(N, 4)

    w1e = enc_w1[:, 0, :].T
    zeh = jnp.zeros((2, H), f32)
    W1e = jnp.concatenate([
        jnp.concatenate([w1e, zeh], axis=1),
        jnp.concatenate([zeh, w1e], axis=1),
    ], axis=0)
    b1e = jnp.tile(enc_b1, 2)[None, :]
    W2e = enc_w2.transpose(2, 1, 0).reshape(2 * H, H)
    w3e = enc_w3[:, :, 0].T
    cbT = codebook.T
    w1d = dec_w1[:, :, ::-1].transpose(0, 2, 1).reshape(D, 2 * H)
    b1d = jnp.tile(dec_b1, 2)[None, :]
    dw2f = dec_w2[:, :, ::-1]
    zer = jnp.zeros((H, H), f32)
    W2d = jnp.concatenate([
        jnp.concatenate([dw2f[:, :, 0], dw2f[:, :, 1], zer, zer], axis=1),
        jnp.concatenate([zer, zer, dw2f[:, :, 0], dw2f[:, :, 1]], axis=1),
    ], axis=0)
    b2d = jnp.tile(dec_b2, 4)[None, :]
    w3v = dec_w3[0, :, 0]
    zv = jnp.zeros((H,), f32)
    W3d = jnp.stack([
        jnp.concatenate([w3v, zv, zv, zv]),
        jnp.concatenate([zv, w3v, zv, zv]),
        jnp.concatenate([zv, zv, w3v, zv]),
        jnp.concatenate([zv, zv, zv, w3v]),
    ], axis=1)
    b3d = dec_b3[None, :]

    wargs = (W1e, b1e, W2e, enc_b2[None, :], w3e, enc_b3[None, :],
             cbT, codebook, w1d, b1d, W2d, b2d, W3d, b3d)

    devs = jax.devices()
    if len(devs) >= 99:
        mesh = jax.sharding.Mesh(devs[:2], ("b",))
        wspecs = tuple(P() for _ in wargs)
        out4, idx, q, commit_acc, recon_acc = jax.shard_map(
            _run_shard, mesh=mesh,
            in_specs=(P("b"),) + wspecs,
            out_specs=(P("b"), P("b"), P("b"), P("b"), P("b")),
            check_vma=False,
        )(img4, *wargs)
        commit_sum = jnp.sum(commit_acc)
        recon_sum = jnp.sum(recon_acc)
    else:
        out4, idx, q, commit_acc, recon_acc = _run_shard(img4, *wargs)
        commit_sum = commit_acc[0, 0]
        recon_sum = recon_acc[0, 0]

    out = out4.reshape(B, T * 4)[:, None, :]
    recon_loss = recon_sum / (B * C * L)
    commit_loss = commit_sum / (B * T * D)
    indices = idx[:, 0].reshape(B, T)
    quantized = q.reshape(B, T, D).transpose(0, 2, 1)
    return (out, recon_loss, commit_loss, indices, quantized)


# speedup vs baseline: 1.1115x; 1.1115x over previous
import jax
import jax.numpy as jnp
from jax.experimental import pallas as pl
from jax.experimental.pallas import tpu as pltpu
from jax.sharding import PartitionSpec as P

B, C, L = 64, 1, 1632
H, D, K = 64, 512, 1024
T = L // 4
N = B * T
BLK = 816


def _vq_kernel(img4_ref, W1e_ref, b1e_ref, W2e_ref, b2e_ref, w3e_ref, b3e_ref,
               cbT_ref, cbh_ref, w1d_ref, b1d_ref, W2d_ref, b2d_ref,
               W3d_ref, b3d_ref, out4_ref, idx_ref, q_ref, commit_ref,
               recon_ref, c2_ref):
    step = pl.program_id(0)

    @pl.when(step == 0)
    def _init():
        cb = cbh_ref[...]
        c2col = jnp.sum(cb * cb, axis=1, keepdims=True)
        c2_ref[...] = c2col.T
        commit_ref[...] = jnp.zeros_like(commit_ref)
        recon_ref[...] = jnp.zeros_like(recon_ref)

    img4 = img4_ref[...]
    h12 = jax.nn.relu(jnp.dot(img4, W1e_ref[...], preferred_element_type=jnp.float32)
                      + b1e_ref[...])
    h2 = jax.nn.relu(jnp.dot(h12, W2e_ref[...], preferred_element_type=jnp.float32)
                     + b2e_ref[...])
    x = jnp.dot(h2, w3e_ref[...], preferred_element_type=jnp.float32) + b3e_ref[...]

    x2 = jnp.sum(x * x, axis=1, keepdims=True)
    scores = jnp.dot(x, cbT_ref[...], preferred_element_type=jnp.float32)
    dist = x2 - 2.0 * scores + c2_ref[...]
    m = jnp.min(dist, axis=1, keepdims=True)
    iota = jax.lax.broadcasted_iota(jnp.int32, dist.shape, 1)
    idx = jnp.min(jnp.where(dist == m, iota, K), axis=1, keepdims=True)
    idx_ref[...] = idx

    onehot = (iota == idx).astype(jnp.float32)
    q = jnp.dot(onehot, cbh_ref[...], preferred_element_type=jnp.float32)
    q_ref[...] = q

    h1d = jax.nn.relu(jnp.dot(q, w1d_ref[...], preferred_element_type=jnp.float32)
                      + b1d_ref[...])
    h2d = jax.nn.relu(jnp.dot(h1d, W2d_ref[...], preferred_element_type=jnp.float32)
                      + b2d_ref[...])
    out4 = jnp.dot(h2d, W3d_ref[...], preferred_element_type=jnp.float32) + b3d_ref[...]
    out4_ref[...] = out4

    commit_ref[...] += jnp.sum(m).reshape(1, 1)
    diff = img4 - out4
    recon_ref[...] += jnp.sum(diff * diff).reshape(1, 1)


def _run_shard(img4, W1e, b1e, W2e, b2e, w3e, b3e, cbT, cbh, w1d, b1d,
               W2d, b2d, W3d, b3d):
    n = img4.shape[0]
    f32 = jnp.float32
    full = lambda shape: pl.BlockSpec(shape, lambda i: tuple(0 for _ in shape))
    return pl.pallas_call(
        _vq_kernel,
        grid=(n // BLK,),
        in_specs=[
            pl.BlockSpec((BLK, 4), lambda i: (i, 0)),
            full((4, 2 * H)), full((1, 2 * H)), full((2 * H, H)), full((1, H)),
            full((H, D)), full((1, D)), full((D, K)), full((K, D)),
            full((D, 2 * H)), full((1, 2 * H)), full((2 * H, 4 * H)),
            full((1, 4 * H)), full((4 * H, 4)), full((1, 1)),
        ],
        out_specs=[
            pl.BlockSpec((BLK, 4), lambda i: (i, 0)),
            pl.BlockSpec((BLK, 1), lambda i: (i, 0)),
            pl.BlockSpec((BLK, D), lambda i: (i, 0)),
            pl.BlockSpec((1, 1), lambda i: (0, 0)),
            pl.BlockSpec((1, 1), lambda i: (0, 0)),
        ],
        out_shape=[
            jax.ShapeDtypeStruct((n, 4), f32),
            jax.ShapeDtypeStruct((n, 1), jnp.int32),
            jax.ShapeDtypeStruct((n, D), f32),
            jax.ShapeDtypeStruct((1, 1), f32),
            jax.ShapeDtypeStruct((1, 1), f32),
        ],
        scratch_shapes=[pltpu.VMEM((1, K), f32)],
    )(img4, W1e, b1e, W2e, b2e, w3e, b3e, cbT, cbh, w1d, b1d, W2d, b2d,
      W3d, b3d)


@jax.jit
def kernel(img, enc_w1, enc_b1, enc_w2, enc_b2, enc_w3, enc_b3, codebook,
           dec_w1, dec_b1, dec_w2, dec_b2, dec_w3, dec_b3):
    f32 = jnp.float32
    img4 = img.reshape(B, T, 4).reshape(N, 4)

    w1e = enc_w1[:, 0, :].T
    zeh = jnp.zeros((2, H), f32)
    W1e = jnp.concatenate([
        jnp.concatenate([w1e, zeh], axis=1),
        jnp.concatenate([zeh, w1e], axis=1),
    ], axis=0)
    b1e = jnp.tile(enc_b1, 2)[None, :]
    W2e = enc_w2.transpose(2, 1, 0).reshape(2 * H, H)
    w3e = enc_w3[:, :, 0].T
    cbT = codebook.T
    w1d = dec_w1[:, :, ::-1].transpose(0, 2, 1).reshape(D, 2 * H)
    b1d = jnp.tile(dec_b1, 2)[None, :]
    dw2f = dec_w2[:, :, ::-1]
    zer = jnp.zeros((H, H), f32)
    W2d = jnp.concatenate([
        jnp.concatenate([dw2f[:, :, 0], dw2f[:, :, 1], zer, zer], axis=1),
        jnp.concatenate([zer, zer, dw2f[:, :, 0], dw2f[:, :, 1]], axis=1),
    ], axis=0)
    b2d = jnp.tile(dec_b2, 4)[None, :]
    w3v = dec_w3[0, :, 0]
    zv = jnp.zeros((H,), f32)
    W3d = jnp.stack([
        jnp.concatenate([w3v, zv, zv, zv]),
        jnp.concatenate([zv, w3v, zv, zv]),
        jnp.concatenate([zv, zv, w3v, zv]),
        jnp.concatenate([zv, zv, zv, w3v]),
    ], axis=1)
    b3d = dec_b3[None, :]

    wargs = (W1e, b1e, W2e, enc_b2[None, :], w3e, enc_b3[None, :],
             cbT, codebook, w1d, b1d, W2d, b2d, W3d, b3d)

    devs = jax.devices()
    if len(devs) >= 99:
        mesh = jax.sharding.Mesh(devs[:2], ("b",))
        wspecs = tuple(P() for _ in wargs)
        out4, idx, q, commit_acc, recon_acc = jax.shard_map(
            _run_shard, mesh=mesh,
            in_specs=(P("b"),) + wspecs,
            out_specs=(P("b"), P("b"), P("b"), P("b"), P("b")),
            check_vma=False,
        )(img4, *wargs)
        commit_sum = jnp.sum(commit_acc)
        recon_sum = jnp.sum(recon_acc)
    else:
        out4, idx, q, commit_acc, recon_acc = _run_shard(img4, *wargs)
        commit_sum = commit_acc[0, 0]
        recon_sum = recon_acc[0, 0]

    out = out4.reshape(B, T * 4)[:, None, :]
    recon_loss = recon_sum / (B * C * L)
    commit_loss = commit_sum / (B * T * D)
    indices = idx[:, 0].reshape(B, T)
    quantized = q.reshape(B, T, D).transpose(0, 2, 1)
    return (out, recon_loss, commit_loss, indices, quantized)


# speedup vs baseline: 1.1824x; 1.0637x over previous
import jax
import jax.numpy as jnp
from jax.experimental import pallas as pl
from jax.experimental.pallas import tpu as pltpu
from jax.sharding import PartitionSpec as P

B, C, L = 64, 1, 1632
H, D, K = 64, 512, 1024
T = L // 4
N = B * T
BLK = 1632


def _vq_kernel(img4_ref, W1e_ref, b1e_ref, W2e_ref, b2e_ref, w3e_ref, b3e_ref,
               cbT_ref, cbh_ref, w1d_ref, b1d_ref, W2d_ref, b2d_ref,
               W3d_ref, b3d_ref, out4_ref, idx_ref, q_ref, commit_ref,
               recon_ref, c2_ref):
    step = pl.program_id(0)

    @pl.when(step == 0)
    def _init():
        cb = cbh_ref[...]
        c2col = jnp.sum(cb * cb, axis=1, keepdims=True)
        c2_ref[...] = c2col.T
        commit_ref[...] = jnp.zeros_like(commit_ref)
        recon_ref[...] = jnp.zeros_like(recon_ref)

    img4 = img4_ref[...]
    h12 = jax.nn.relu(jnp.dot(img4, W1e_ref[...], preferred_element_type=jnp.float32)
                      + b1e_ref[...])
    h2 = jax.nn.relu(jnp.dot(h12, W2e_ref[...], preferred_element_type=jnp.float32)
                     + b2e_ref[...])
    x = jnp.dot(h2, w3e_ref[...], preferred_element_type=jnp.float32) + b3e_ref[...]

    x2 = jnp.sum(x * x, axis=1, keepdims=True)
    scores = jnp.dot(x, cbT_ref[...], preferred_element_type=jnp.float32)
    dist = x2 - 2.0 * scores + c2_ref[...]
    m = jnp.min(dist, axis=1, keepdims=True)
    iota = jax.lax.broadcasted_iota(jnp.int32, dist.shape, 1)
    idx = jnp.min(jnp.where(dist == m, iota, K), axis=1, keepdims=True)
    idx_ref[...] = idx

    onehot = (iota == idx).astype(jnp.float32)
    q = jnp.dot(onehot, cbh_ref[...], preferred_element_type=jnp.float32)
    q_ref[...] = q

    h1d = jax.nn.relu(jnp.dot(q, w1d_ref[...], preferred_element_type=jnp.float32)
                      + b1d_ref[...])
    h2d = jax.nn.relu(jnp.dot(h1d, W2d_ref[...], preferred_element_type=jnp.float32)
                      + b2d_ref[...])
    out4 = jnp.dot(h2d, W3d_ref[...], preferred_element_type=jnp.float32) + b3d_ref[...]
    out4_ref[...] = out4

    commit_ref[...] += jnp.sum(m).reshape(1, 1)
    diff = img4 - out4
    recon_ref[...] += jnp.sum(diff * diff).reshape(1, 1)


def _run_shard(img4, W1e, b1e, W2e, b2e, w3e, b3e, cbT, cbh, w1d, b1d,
               W2d, b2d, W3d, b3d):
    n = img4.shape[0]
    f32 = jnp.float32
    full = lambda shape: pl.BlockSpec(shape, lambda i: tuple(0 for _ in shape))
    return pl.pallas_call(
        _vq_kernel,
        grid=(n // BLK,),
        in_specs=[
            pl.BlockSpec((BLK, 4), lambda i: (i, 0)),
            full((4, 2 * H)), full((1, 2 * H)), full((2 * H, H)), full((1, H)),
            full((H, D)), full((1, D)), full((D, K)), full((K, D)),
            full((D, 2 * H)), full((1, 2 * H)), full((2 * H, 4 * H)),
            full((1, 4 * H)), full((4 * H, 4)), full((1, 1)),
        ],
        out_specs=[
            pl.BlockSpec((BLK, 4), lambda i: (i, 0)),
            pl.BlockSpec((BLK, 1), lambda i: (i, 0)),
            pl.BlockSpec((BLK, D), lambda i: (i, 0)),
            pl.BlockSpec((1, 1), lambda i: (0, 0)),
            pl.BlockSpec((1, 1), lambda i: (0, 0)),
        ],
        out_shape=[
            jax.ShapeDtypeStruct((n, 4), f32),
            jax.ShapeDtypeStruct((n, 1), jnp.int32),
            jax.ShapeDtypeStruct((n, D), f32),
            jax.ShapeDtypeStruct((1, 1), f32),
            jax.ShapeDtypeStruct((1, 1), f32),
        ],
        scratch_shapes=[pltpu.VMEM((1, K), f32)],
    )(img4, W1e, b1e, W2e, b2e, w3e, b3e, cbT, cbh, w1d, b1d, W2d, b2d,
      W3d, b3d)


@jax.jit
def kernel(img, enc_w1, enc_b1, enc_w2, enc_b2, enc_w3, enc_b3, codebook,
           dec_w1, dec_b1, dec_w2, dec_b2, dec_w3, dec_b3):
    f32 = jnp.float32
    img4 = img.reshape(B, T, 4).reshape(N, 4)

    w1e = enc_w1[:, 0, :].T
    zeh = jnp.zeros((2, H), f32)
    W1e = jnp.concatenate([
        jnp.concatenate([w1e, zeh], axis=1),
        jnp.concatenate([zeh, w1e], axis=1),
    ], axis=0)
    b1e = jnp.tile(enc_b1, 2)[None, :]
    W2e = enc_w2.transpose(2, 1, 0).reshape(2 * H, H)
    w3e = enc_w3[:, :, 0].T
    cbT = codebook.T
    w1d = dec_w1[:, :, ::-1].transpose(0, 2, 1).reshape(D, 2 * H)
    b1d = jnp.tile(dec_b1, 2)[None, :]
    dw2f = dec_w2[:, :, ::-1]
    zer = jnp.zeros((H, H), f32)
    W2d = jnp.concatenate([
        jnp.concatenate([dw2f[:, :, 0], dw2f[:, :, 1], zer, zer], axis=1),
        jnp.concatenate([zer, zer, dw2f[:, :, 0], dw2f[:, :, 1]], axis=1),
    ], axis=0)
    b2d = jnp.tile(dec_b2, 4)[None, :]
    w3v = dec_w3[0, :, 0]
    zv = jnp.zeros((H,), f32)
    W3d = jnp.stack([
        jnp.concatenate([w3v, zv, zv, zv]),
        jnp.concatenate([zv, w3v, zv, zv]),
        jnp.concatenate([zv, zv, w3v, zv]),
        jnp.concatenate([zv, zv, zv, w3v]),
    ], axis=1)
    b3d = dec_b3[None, :]

    wargs = (W1e, b1e, W2e, enc_b2[None, :], w3e, enc_b3[None, :],
             cbT, codebook, w1d, b1d, W2d, b2d, W3d, b3d)

    devs = jax.devices()
    if len(devs) >= 99:
        mesh = jax.sharding.Mesh(devs[:2], ("b",))
        wspecs = tuple(P() for _ in wargs)
        out4, idx, q, commit_acc, recon_acc = jax.shard_map(
            _run_shard, mesh=mesh,
            in_specs=(P("b"),) + wspecs,
            out_specs=(P("b"), P("b"), P("b"), P("b"), P("b")),
            check_vma=False,
        )(img4, *wargs)
        commit_sum = jnp.sum(commit_acc)
        recon_sum = jnp.sum(recon_acc)
    else:
        out4, idx, q, commit_acc, recon_acc = _run_shard(img4, *wargs)
        commit_sum = commit_acc[0, 0]
        recon_sum = recon_acc[0, 0]

    out = out4.reshape(B, T * 4)[:, None, :]
    recon_loss = recon_sum / (B * C * L)
    commit_loss = commit_sum / (B * T * D)
    indices = idx[:, 0].reshape(B, T)
    quantized = q.reshape(B, T, D).transpose(0, 2, 1)
    return (out, recon_loss, commit_loss, indices, quantized)


# speedup vs baseline: 1.1913x; 1.0076x over previous
import jax
import jax.numpy as jnp
from jax.experimental import pallas as pl
from jax.experimental.pallas import tpu as pltpu
from jax.sharding import PartitionSpec as P

B, C, L = 64, 1, 1632
H, D, K = 64, 512, 1024
T = L // 4
N = B * T
BLK = 3264


def _vq_kernel(img4_ref, W1e_ref, b1e_ref, W2e_ref, b2e_ref, w3e_ref, b3e_ref,
               cbT_ref, cbh_ref, w1d_ref, b1d_ref, W2d_ref, b2d_ref,
               W3d_ref, b3d_ref, out4_ref, idx_ref, q_ref, commit_ref,
               recon_ref, c2_ref):
    step = pl.program_id(0)

    @pl.when(step == 0)
    def _init():
        cb = cbh_ref[...]
        c2col = jnp.sum(cb * cb, axis=1, keepdims=True)
        c2_ref[...] = c2col.T
        commit_ref[...] = jnp.zeros_like(commit_ref)
        recon_ref[...] = jnp.zeros_like(recon_ref)

    img4 = img4_ref[...]
    h12 = jax.nn.relu(jnp.dot(img4, W1e_ref[...], preferred_element_type=jnp.float32)
                      + b1e_ref[...])
    h2 = jax.nn.relu(jnp.dot(h12, W2e_ref[...], preferred_element_type=jnp.float32)
                     + b2e_ref[...])
    x = jnp.dot(h2, w3e_ref[...], preferred_element_type=jnp.float32) + b3e_ref[...]

    x2 = jnp.sum(x * x, axis=1, keepdims=True)
    scores = jnp.dot(x, cbT_ref[...], preferred_element_type=jnp.float32)
    dist = x2 - 2.0 * scores + c2_ref[...]
    m = jnp.min(dist, axis=1, keepdims=True)
    iota = jax.lax.broadcasted_iota(jnp.int32, dist.shape, 1)
    idx = jnp.min(jnp.where(dist == m, iota, K), axis=1, keepdims=True)
    idx_ref[...] = idx

    onehot = (iota == idx).astype(jnp.float32)
    q = jnp.dot(onehot, cbh_ref[...], preferred_element_type=jnp.float32)
    q_ref[...] = q

    h1d = jax.nn.relu(jnp.dot(q, w1d_ref[...], preferred_element_type=jnp.float32)
                      + b1d_ref[...])
    h2d = jax.nn.relu(jnp.dot(h1d, W2d_ref[...], preferred_element_type=jnp.float32)
                      + b2d_ref[...])
    out4 = jnp.dot(h2d, W3d_ref[...], preferred_element_type=jnp.float32) + b3d_ref[...]
    out4_ref[...] = out4

    commit_ref[...] += jnp.sum(m).reshape(1, 1)
    diff = img4 - out4
    recon_ref[...] += jnp.sum(diff * diff).reshape(1, 1)


def _run_shard(img4, W1e, b1e, W2e, b2e, w3e, b3e, cbT, cbh, w1d, b1d,
               W2d, b2d, W3d, b3d):
    n = img4.shape[0]
    f32 = jnp.float32
    full = lambda shape: pl.BlockSpec(shape, lambda i: tuple(0 for _ in shape))
    return pl.pallas_call(
        _vq_kernel,
        grid=(n // BLK,),
        in_specs=[
            pl.BlockSpec((BLK, 4), lambda i: (i, 0)),
            full((4, 2 * H)), full((1, 2 * H)), full((2 * H, H)), full((1, H)),
            full((H, D)), full((1, D)), full((D, K)), full((K, D)),
            full((D, 2 * H)), full((1, 2 * H)), full((2 * H, 4 * H)),
            full((1, 4 * H)), full((4 * H, 4)), full((1, 1)),
        ],
        out_specs=[
            pl.BlockSpec((BLK, 4), lambda i: (i, 0)),
            pl.BlockSpec((BLK, 1), lambda i: (i, 0)),
            pl.BlockSpec((BLK, D), lambda i: (i, 0)),
            pl.BlockSpec((1, 1), lambda i: (0, 0)),
            pl.BlockSpec((1, 1), lambda i: (0, 0)),
        ],
        out_shape=[
            jax.ShapeDtypeStruct((n, 4), f32),
            jax.ShapeDtypeStruct((n, 1), jnp.int32),
            jax.ShapeDtypeStruct((n, D), f32),
            jax.ShapeDtypeStruct((1, 1), f32),
            jax.ShapeDtypeStruct((1, 1), f32),
        ],
        scratch_shapes=[pltpu.VMEM((1, K), f32)],
    )(img4, W1e, b1e, W2e, b2e, w3e, b3e, cbT, cbh, w1d, b1d, W2d, b2d,
      W3d, b3d)


@jax.jit
def kernel(img, enc_w1, enc_b1, enc_w2, enc_b2, enc_w3, enc_b3, codebook,
           dec_w1, dec_b1, dec_w2, dec_b2, dec_w3, dec_b3):
    f32 = jnp.float32
    img4 = img.reshape(B, T, 4).reshape(N, 4)

    w1e = enc_w1[:, 0, :].T
    zeh = jnp.zeros((2, H), f32)
    W1e = jnp.concatenate([
        jnp.concatenate([w1e, zeh], axis=1),
        jnp.concatenate([zeh, w1e], axis=1),
    ], axis=0)
    b1e = jnp.tile(enc_b1, 2)[None, :]
    W2e = enc_w2.transpose(2, 1, 0).reshape(2 * H, H)
    w3e = enc_w3[:, :, 0].T
    cbT = codebook.T
    w1d = dec_w1[:, :, ::-1].transpose(0, 2, 1).reshape(D, 2 * H)
    b1d = jnp.tile(dec_b1, 2)[None, :]
    dw2f = dec_w2[:, :, ::-1]
    zer = jnp.zeros((H, H), f32)
    W2d = jnp.concatenate([
        jnp.concatenate([dw2f[:, :, 0], dw2f[:, :, 1], zer, zer], axis=1),
        jnp.concatenate([zer, zer, dw2f[:, :, 0], dw2f[:, :, 1]], axis=1),
    ], axis=0)
    b2d = jnp.tile(dec_b2, 4)[None, :]
    w3v = dec_w3[0, :, 0]
    zv = jnp.zeros((H,), f32)
    W3d = jnp.stack([
        jnp.concatenate([w3v, zv, zv, zv]),
        jnp.concatenate([zv, w3v, zv, zv]),
        jnp.concatenate([zv, zv, w3v, zv]),
        jnp.concatenate([zv, zv, zv, w3v]),
    ], axis=1)
    b3d = dec_b3[None, :]

    wargs = (W1e, b1e, W2e, enc_b2[None, :], w3e, enc_b3[None, :],
             cbT, codebook, w1d, b1d, W2d, b2d, W3d, b3d)

    devs = jax.devices()
    if len(devs) >= 99:
        mesh = jax.sharding.Mesh(devs[:2], ("b",))
        wspecs = tuple(P() for _ in wargs)
        out4, idx, q, commit_acc, recon_acc = jax.shard_map(
            _run_shard, mesh=mesh,
            in_specs=(P("b"),) + wspecs,
            out_specs=(P("b"), P("b"), P("b"), P("b"), P("b")),
            check_vma=False,
        )(img4, *wargs)
        commit_sum = jnp.sum(commit_acc)
        recon_sum = jnp.sum(recon_acc)
    else:
        out4, idx, q, commit_acc, recon_acc = _run_shard(img4, *wargs)
        commit_sum = commit_acc[0, 0]
        recon_sum = recon_acc[0, 0]

    out = out4.reshape(B, T * 4)[:, None, :]
    recon_loss = recon_sum / (B * C * L)
    commit_loss = commit_sum / (B * T * D)
    indices = idx[:, 0].reshape(B, T)
    quantized = q.reshape(B, T, D).transpose(0, 2, 1)
    return (out, recon_loss, commit_loss, indices, quantized)
